# Initial kernel scaffold; baseline (speedup 1.0000x reference)
import functools

import jax
import jax.numpy as jnp
from jax import lax
from jax.experimental import pallas as pl
from jax.experimental.pallas import tpu as pltpu
from jax.experimental.pallas import tpu_sc as plsc

N = 10000
E = 320000
D = 128
NET = 7
NC = 2
NS = 16
NW = NC * NS
L = 16

EPT = E // NW
EPC = E // NS
CH = 80
NCH2 = EPT // CH
NCH1 = EPC // CH
NB = 70144
SLC = NB // NS
ROWS_PER_TILE = N // NS
DR = 125


def _ytab_body(x_ref, w_ref, o_ref):
    o_ref[...] = jnp.dot(x_ref[...], w_ref[...],
                         preferred_element_type=jnp.float32)


def _make_ytab(x, weights):
    return pl.pallas_call(
        _ytab_body,
        grid=(N // 500, NET),
        in_specs=[
            pl.BlockSpec((500, D), lambda i, e: (i, 0)),
            pl.BlockSpec((D, D), lambda i, e: (e, 0)),
        ],
        out_specs=pl.BlockSpec((500, D), lambda i, e: (e * (N // 500) + i, 0)),
        out_shape=jax.ShapeDtypeStruct((NET * N, D), jnp.float32),
    )(x, weights)


def _final_body(a_ref, b_ref, c_ref, o_ref):
    o_ref[...] = a_ref[...] + b_ref[...] + c_ref[...]


def _final_add(p0, p1, y6):
    return pl.pallas_call(
        _final_body,
        grid=(N // 500,),
        in_specs=[pl.BlockSpec((500, D), lambda i: (i, 0))] * 3,
        out_specs=pl.BlockSpec((500, D), lambda i: (i, 0)),
        out_shape=jax.ShapeDtypeStruct((N, D), jnp.float32),
    )(p0, p1, y6)


def _sc_body(ytab, col, row, et, parts,
             scale_sh, acc_sh,
             rows, colv, rowv, etv, tblv, bktv, sclv, onesv, cntv, zbuf,
             sem):
    core = lax.axis_index("c")
    sid = lax.axis_index("s")
    wid = sid * NC + core

    def _z16(i, _):
        zbuf[pl.ds(i * L, L)] = jnp.zeros((L,), jnp.float32)
        return _
    lax.fori_loop(0, (DR * D) // L, _z16, None)
    pltpu.sync_copy(zbuf.at[pl.ds(0, SLC)], scale_sh.at[pl.ds(sid * SLC, SLC)])
    zmat = zbuf.reshape(DR, D)
    for b in range(ROWS_PER_TILE // DR):
        pltpu.sync_copy(
            zmat, acc_sh.at[pl.ds(sid * ROWS_PER_TILE + b * DR, DR), :])
    def _ones16(i, _):
        onesv[pl.ds(i * L, L)] = jnp.ones((L,), jnp.float32)
        return _
    lax.fori_loop(0, CH // L, _ones16, None)
    plsc.subcore_barrier()

    def _cnt_chunk(i, _):
        base = sid * EPC + i * CH
        pltpu.sync_copy(row.at[pl.ds(base, CH)], rowv)
        pltpu.sync_copy(et.at[pl.ds(base, CH)], etv)
        for k in range(CH // L):
            sl = pl.ds(k * L, L)
            bktv[sl] = rowv[sl] * NET + etv[sl]
        pltpu.sync_copy(onesv, scale_sh.at[bktv], add=True)
        return _
    lax.fori_loop(0, NCH1, _cnt_chunk, None)
    plsc.subcore_barrier()

    pltpu.sync_copy(scale_sh.at[pl.ds(sid * SLC, SLC)], cntv)
    def _scale16(i, _):
        sl = pl.ds(i * L, L)
        b = sid * SLC + i * L + lax.iota(jnp.int32, L)
        c = cntv[sl]
        s = 1.0 / jnp.maximum(c, 1.0)
        cntv[sl] = jnp.where(b % NET == NET - 1, 0.0, s)
        return _
    lax.fori_loop(0, SLC // L, _scale16, None)
    pltpu.sync_copy(cntv, scale_sh.at[pl.ds(sid * SLC, SLC)])
    plsc.subcore_barrier()

    def _edge_chunk(i, _):
        base = wid * EPT + i * CH
        pltpu.sync_copy(col.at[pl.ds(base, CH)], colv)
        pltpu.sync_copy(row.at[pl.ds(base, CH)], rowv)
        pltpu.sync_copy(et.at[pl.ds(base, CH)], etv)
        for k in range(CH // L):
            sl = pl.ds(k * L, L)
            e16 = etv[sl]
            tblv[sl] = e16 * N + colv[sl]
            bktv[sl] = rowv[sl] * NET + e16
        pltpu.sync_copy(scale_sh.at[bktv], sclv)
        pltpu.async_copy(ytab.at[tblv], rows, sem).wait()

        def _mul(e, _):
            sv = lax.broadcast(sclv[e], (L,))
            for j in range(D // L):
                sl = pl.ds(j * L, L)
                rows[e, sl] = rows[e, sl] * sv
            return _
        lax.fori_loop(0, CH, _mul, None)
        pltpu.sync_copy(rows, acc_sh.at[rowv], add=True)
        return _
    lax.fori_loop(0, NCH2, _edge_chunk, None)
    plsc.subcore_barrier()

    for b in range(ROWS_PER_TILE // DR):
        off = sid * ROWS_PER_TILE + b * DR
        pltpu.sync_copy(acc_sh.at[pl.ds(off, DR), :], zmat)
        pltpu.sync_copy(zmat, parts.at[core, pl.ds(off, DR), :])


def _make_sc(ytab, col, row, et):
    mesh = plsc.VectorSubcoreMesh(core_axis_name="c", subcore_axis_name="s")
    f = pl.kernel(
        _sc_body,
        out_type=jax.ShapeDtypeStruct((NC, N, D), jnp.float32),
        mesh=mesh,
        scratch_types=[
            pltpu.VMEM_SHARED((NB,), jnp.float32),
            pltpu.VMEM_SHARED((N, D), jnp.float32),
            pltpu.VMEM((CH, D), jnp.float32),
            pltpu.VMEM((CH,), jnp.int32),
            pltpu.VMEM((CH,), jnp.int32),
            pltpu.VMEM((CH,), jnp.int32),
            pltpu.VMEM((CH,), jnp.int32),
            pltpu.VMEM((CH,), jnp.int32),
            pltpu.VMEM((CH,), jnp.float32),
            pltpu.VMEM((CH,), jnp.float32),
            pltpu.VMEM((SLC,), jnp.float32),
            pltpu.VMEM((DR * D,), jnp.float32),
            pltpu.SemaphoreType.DMA,
        ],
    )
    return f(ytab, col, row, et)


def kernel(x, edge_index, edge_type, weights):
    ytab = _make_ytab(x, weights)
    row = edge_index[0]
    col = edge_index[1]
    parts = _make_sc(ytab, col, row, edge_type)
    return _final_add(parts[0], parts[1], ytab[(NET - 1) * N:])


# speedup vs baseline: 4.9033x; 4.9033x over previous
import jax
import jax.numpy as jnp
from jax import lax
from jax.experimental import pallas as pl
from jax.experimental.pallas import tpu as pltpu
from jax.experimental.pallas import tpu_sc as plsc

N = 10000
E = 320000
D = 128
NET = 7
NC = 2
NS = 16
NW = NC * NS
L = 16

CH = 80
EPT = E // NW
SEG = 2000
NSEG = EPT // SEG
CPS = SEG // CH
EPC = E // NS
SCE = 400
KPS = SCE // CH
NSC1 = EPC // SCE
NB = 70144
SLC = NB // NS
NPAD = 10240
RPT = NPAD // NS
DRB = RPT // CH


def _ytab_body(x_ref, w_ref, o_ref):
    o_ref[...] = jnp.dot(x_ref[...], w_ref[...],
                         preferred_element_type=jnp.float32)


def _make_ytab(x, weights):
    return pl.pallas_call(
        _ytab_body,
        grid=(N // 1000, NET),
        in_specs=[
            pl.BlockSpec((1000, D), lambda i, e: (i, 0)),
            pl.BlockSpec((D, D), lambda i, e: (e, 0)),
        ],
        out_specs=pl.BlockSpec((1000, D), lambda i, e: (e * (N // 1000) + i, 0)),
        out_shape=jax.ShapeDtypeStruct((NET * N, D), jnp.float32),
    )(x, weights)


def _final_body(a_ref, b_ref, c_ref, o_ref):
    o_ref[...] = a_ref[...] + b_ref[...] + c_ref[...]


def _final_add(p0, p1, y6):
    return pl.pallas_call(
        _final_body,
        grid=(N // 1000,),
        in_specs=[pl.BlockSpec((1000, D), lambda i: (i, 0))] * 3,
        out_specs=pl.BlockSpec((1000, D), lambda i: (i, 0)),
        out_shape=jax.ShapeDtypeStruct((N, D), jnp.float32),
    )(p0, p1, y6)


def _sc_body(ytab, col, row, et, parts,
             scale_sh, acc_sh,
             rows2, tblS, bktS, dstS, scl2,
             rowE, etE, bktC, dstCa, dstCb, onesv, cntv,
             sem_g, sem_s):
    core = lax.axis_index("c")
    sid = lax.axis_index("s")
    wid = sid * NC + core

    def _zc16(i, _):
        cntv[pl.ds(i * L, L)] = jnp.zeros((L,), jnp.float32)
        return _
    lax.fori_loop(0, SLC // L, _zc16, None)
    pltpu.sync_copy(cntv, scale_sh.at[pl.ds(sid * SLC, SLC)])

    def _zm(r, _):
        for j in range(D // L):
            rows2[0, r, pl.ds(j * L, L)] = jnp.zeros((L,), jnp.float32)
        return _
    lax.fori_loop(0, CH, _zm, None)
    for b in range(DRB):
        pltpu.sync_copy(rows2.at[0],
                        acc_sh.at[pl.ds(sid * RPT + b * CH, CH), :])

    def _ones16(i, _):
        onesv[pl.ds(i * L, L)] = jnp.ones((L,), jnp.float32)
        return _
    lax.fori_loop(0, CH // L, _ones16, None)
    plsc.subcore_barrier()

    base1 = sid * EPC

    def _p1_iter(s, _):
        off = pl.ds(base1 + s * SCE, SCE)
        pltpu.sync_copy(row.at[off], rowE)
        pltpu.sync_copy(et.at[off], etE)
        for k in range(KPS):
            for m in range(CH // L):
                sl = pl.ds(k * CH + m * L, L)
                bktC[pl.ds(m * L, L)] = rowE[sl] * NET + etE[sl]
            pltpu.sync_copy(onesv, scale_sh.at[bktC], add=True)
        return _
    lax.fori_loop(0, NSC1, _p1_iter, None)
    plsc.subcore_barrier()

    pltpu.sync_copy(scale_sh.at[pl.ds(sid * SLC, SLC)], cntv)

    def _scale16(i, _):
        sl = pl.ds(i * L, L)
        b = sid * SLC + i * L + lax.iota(jnp.int32, L)
        c = cntv[sl]
        s = 1.0 / jnp.maximum(c, 1.0)
        cntv[sl] = jnp.where(b % NET == NET - 1, 0.0, s)
        return _
    lax.fori_loop(0, SLC // L, _scale16, None)
    pltpu.sync_copy(cntv, scale_sh.at[pl.ds(sid * SLC, SLC)])
    plsc.subcore_barrier()

    base2 = wid * EPT

    def _seg_iter(g, _):
        off = pl.ds(base2 + g * SEG, SEG)
        pltpu.sync_copy(col.at[off], tblS)
        pltpu.sync_copy(row.at[off], dstS)
        pltpu.sync_copy(et.at[off], bktS)

        def _cg(m, _):
            sl = pl.ds(m * L, L)
            e16 = bktS[sl]
            tblS[sl] = e16 * N + tblS[sl]
            bktS[sl] = dstS[sl] * NET + e16
            return _
        lax.fori_loop(0, SEG // L, _cg, None)

        def _sg(k, _):
            ck = pl.ds(k * CH, CH)
            pltpu.sync_copy(scale_sh.at[bktS.at[ck]], scl2.at[ck])
            return _
        lax.fori_loop(0, CPS, _sg, None)

        def _chunk_iter(i, _):
            ik = pl.ds(i * CH, CH)
            pltpu.async_copy(ytab.at[tblS.at[ik]], rows2.at[0],
                             sem_g).wait()
            for m in range(CH // L):
                dstCa[pl.ds(m * L, L)] = dstS[pl.ds(i * CH + m * L, L)]

            def _mul(u, _):
                for v in range(2):
                    e = u * 2 + v
                    sv = lax.broadcast(scl2[pl.ds(i * CH + e, L)][0], (L,))
                    for j in range(D // L):
                        sl = pl.ds(j * L, L)
                        rows2[0, e, sl] = rows2[0, e, sl] * sv
                return _
            lax.fori_loop(0, CH // 2, _mul, None)
            pltpu.sync_copy(rows2.at[0], acc_sh.at[dstCa], add=True)
            return _
        lax.fori_loop(0, CPS, _chunk_iter, None)
        return _
    lax.fori_loop(0, NSEG, _seg_iter, None)
    plsc.subcore_barrier()

    for b in range(DRB):
        off = sid * RPT + b * CH
        pltpu.sync_copy(acc_sh.at[pl.ds(off, CH), :], rows2.at[0])
        pltpu.sync_copy(rows2.at[0], parts.at[core, pl.ds(off, CH), :])


def _make_sc(ytab, col, row, et):
    mesh = plsc.VectorSubcoreMesh(core_axis_name="c", subcore_axis_name="s",
                                  num_cores=NC, num_subcores=NS)
    f = pl.kernel(
        _sc_body,
        out_type=jax.ShapeDtypeStruct((NC, NPAD, D), jnp.float32),
        mesh=mesh,
        scratch_types=[
            pltpu.VMEM_SHARED((NB,), jnp.float32),
            pltpu.VMEM_SHARED((NPAD, D), jnp.float32),
            pltpu.VMEM((2, CH, D), jnp.float32),
            pltpu.VMEM((SEG,), jnp.int32),
            pltpu.VMEM((SEG,), jnp.int32),
            pltpu.VMEM((SEG,), jnp.int32),
            pltpu.VMEM((SEG + L,), jnp.float32),
            pltpu.VMEM((SCE,), jnp.int32),
            pltpu.VMEM((SCE,), jnp.int32),
            pltpu.VMEM((CH,), jnp.int32),
            pltpu.VMEM((CH,), jnp.int32),
            pltpu.VMEM((CH,), jnp.int32),
            pltpu.VMEM((CH,), jnp.float32),
            pltpu.VMEM((SLC,), jnp.float32),
            pltpu.SemaphoreType.DMA,
            pltpu.SemaphoreType.DMA,
        ],
    )
    return f(ytab, col, row, et)


def kernel(x, edge_index, edge_type, weights):
    ytab = _make_ytab(x, weights)
    row = edge_index[0]
    col = edge_index[1]
    parts = _make_sc(ytab, col, row, edge_type)
    return _final_add(parts[0, :N], parts[1, :N], ytab[(NET - 1) * N:])


# speedup vs baseline: 5.8618x; 1.1955x over previous
import jax
import jax.numpy as jnp
from jax import lax
from jax.experimental import pallas as pl
from jax.experimental.pallas import tpu as pltpu
from jax.experimental.pallas import tpu_sc as plsc

N = 10000
E = 320000
D = 128
NET = 7
NC = 2
NS = 16
NW = NC * NS
L = 16

CH = 80
EPT = E // NW
SEG = 2000
NSEG = EPT // SEG
CPS = SEG // CH
EPC = E // NS
SCE = 2000
KPS = SCE // CH
NSC1 = EPC // SCE
NB = 70144
SLC = NB // NS
NPAD = 10240
RPT = NPAD // NS
DRB = RPT // CH


def _ytab_body(x_ref, w_ref, o_ref):
    o_ref[...] = jnp.dot(x_ref[...], w_ref[...],
                         preferred_element_type=jnp.float32)


def _make_ytab(x, weights):
    return pl.pallas_call(
        _ytab_body,
        grid=(N // 1000, NET),
        in_specs=[
            pl.BlockSpec((1000, D), lambda i, e: (i, 0)),
            pl.BlockSpec((D, D), lambda i, e: (e, 0)),
        ],
        out_specs=pl.BlockSpec((1000, D), lambda i, e: (e * (N // 1000) + i, 0)),
        out_shape=jax.ShapeDtypeStruct((NET * N, D), jnp.float32),
    )(x, weights)


def _final_body(a_ref, b_ref, c_ref, o_ref):
    o_ref[...] = a_ref[...] + b_ref[...] + c_ref[...]


def _final_add(p0, p1, y6):
    return pl.pallas_call(
        _final_body,
        grid=(N // 1000,),
        in_specs=[pl.BlockSpec((1000, D), lambda i: (i, 0))] * 3,
        out_specs=pl.BlockSpec((1000, D), lambda i: (i, 0)),
        out_shape=jax.ShapeDtypeStruct((N, D), jnp.float32),
    )(p0, p1, y6)


def _sc_body(ytab, col, row, et, parts,
             scale_sh, acc_sh,
             rows2, tblS, bktS, dstS, scl2,
             rowE, etE, bktC, dstCa, dstCb, onesv, cntv,
             sem_g, sem_s):
    core = lax.axis_index("c")
    sid = lax.axis_index("s")
    wid = sid * NC + core

    def _zc16(i, _):
        cntv[pl.ds(i * L, L)] = jnp.zeros((L,), jnp.float32)
        return _
    lax.fori_loop(0, SLC // L, _zc16, None)
    pltpu.sync_copy(cntv, scale_sh.at[pl.ds(sid * SLC, SLC)])

    def _zm(r, _):
        for j in range(D // L):
            rows2[0, r, pl.ds(j * L, L)] = jnp.zeros((L,), jnp.float32)
        return _
    lax.fori_loop(0, CH, _zm, None)
    for b in range(DRB):
        pltpu.sync_copy(rows2.at[0],
                        acc_sh.at[pl.ds(sid * RPT + b * CH, CH), :])

    def _ones16(i, _):
        onesv[pl.ds(i * L, L)] = jnp.ones((L,), jnp.float32)
        return _
    lax.fori_loop(0, CH // L, _ones16, None)
    plsc.subcore_barrier()

    base1 = sid * EPC

    def _p1_iter(s, _):
        off = pl.ds(base1 + s * SCE, SCE)
        pltpu.sync_copy(row.at[off], rowE)
        pltpu.sync_copy(et.at[off], etE)
        def _p1_chunk(k, _):
            def _bk(m, _):
                bktC[pl.ds(m * L, L)] = (rowE[pl.ds(k * CH + m * L, L)] * NET
                                         + etE[pl.ds(k * CH + m * L, L)])
                return _
            lax.fori_loop(0, CH // L, _bk, None)
            pltpu.sync_copy(onesv, scale_sh.at[bktC], add=True)
            return _
        lax.fori_loop(0, KPS, _p1_chunk, None)
        return _
    lax.fori_loop(0, NSC1, _p1_iter, None)
    plsc.subcore_barrier()

    pltpu.sync_copy(scale_sh.at[pl.ds(sid * SLC, SLC)], cntv)

    def _scale16(i, _):
        sl = pl.ds(i * L, L)
        b = sid * SLC + i * L + lax.iota(jnp.int32, L)
        c = cntv[sl]
        s = 1.0 / jnp.maximum(c, 1.0)
        cntv[sl] = jnp.where(b % NET == NET - 1, 0.0, s)
        return _
    lax.fori_loop(0, SLC // L, _scale16, None)
    pltpu.sync_copy(cntv, scale_sh.at[pl.ds(sid * SLC, SLC)])
    plsc.subcore_barrier()

    base2 = wid * EPT

    def _seg_iter(g, _):
        off = pl.ds(base2 + g * SEG, SEG)
        pltpu.sync_copy(col.at[off], tblS)
        pltpu.sync_copy(row.at[off], dstS)
        pltpu.sync_copy(et.at[off], bktS)

        def _cg(m, _):
            sl = pl.ds(m * L, L)
            e16 = bktS[sl]
            tblS[sl] = e16 * N + tblS[sl]
            bktS[sl] = dstS[sl] * NET + e16
            return _
        lax.fori_loop(0, SEG // L, _cg, None)

        def _sg(k, _):
            ck = pl.ds(k * CH, CH)
            pltpu.sync_copy(scale_sh.at[bktS.at[ck]], scl2.at[ck])
            return _
        lax.fori_loop(0, CPS, _sg, None)

        def _do_chunk(i, p, dstC):
            for m in range(CH // L):
                dstC[pl.ds(m * L, L)] = dstS[pl.ds(i * CH + m * L, L)]

            def _mul(u, _):
                for v in range(4):
                    e = u * 4 + v
                    sv = lax.broadcast(scl2[pl.ds(i * CH + e, L)][0], (L,))
                    for j in range(D // L):
                        sl = pl.ds(j * L, L)
                        rows2[p, e, sl] = rows2[p, e, sl] * sv
                return _
            lax.fori_loop(0, CH // 4, _mul, None)
            pltpu.sync_copy(rows2.at[p], acc_sh.at[dstC], add=True)

        def _pair_iter(u, _):
            a = 2 * u
            b = a + 1
            ga = pltpu.async_copy(ytab.at[tblS.at[pl.ds(a * CH, CH)]],
                                  rows2.at[0], sem_g)
            gb = pltpu.async_copy(ytab.at[tblS.at[pl.ds(b * CH, CH)]],
                                  rows2.at[1], sem_s)
            ga.wait()
            _do_chunk(a, 0, dstCa)
            gb.wait()
            _do_chunk(b, 1, dstCb)
            return _
        lax.fori_loop(0, CPS // 2, _pair_iter, None)
        gl = pltpu.async_copy(ytab.at[tblS.at[pl.ds((CPS - 1) * CH, CH)]],
                              rows2.at[0], sem_g)
        gl.wait()
        _do_chunk(CPS - 1, 0, dstCa)
        return _
    lax.fori_loop(0, NSEG, _seg_iter, None)
    plsc.subcore_barrier()

    for b in range(DRB):
        off = sid * RPT + b * CH
        pltpu.sync_copy(acc_sh.at[pl.ds(off, CH), :], rows2.at[0])
        pltpu.sync_copy(rows2.at[0], parts.at[core, pl.ds(off, CH), :])


def _make_sc(ytab, col, row, et):
    mesh = plsc.VectorSubcoreMesh(core_axis_name="c", subcore_axis_name="s",
                                  num_cores=NC, num_subcores=NS)
    f = pl.kernel(
        _sc_body,
        out_type=jax.ShapeDtypeStruct((NC, NPAD, D), jnp.float32),
        mesh=mesh,
        scratch_types=[
            pltpu.VMEM_SHARED((NB,), jnp.float32),
            pltpu.VMEM_SHARED((NPAD, D), jnp.float32),
            pltpu.VMEM((2, CH, D), jnp.float32),
            pltpu.VMEM((SEG,), jnp.int32),
            pltpu.VMEM((SEG,), jnp.int32),
            pltpu.VMEM((SEG,), jnp.int32),
            pltpu.VMEM((SEG + L,), jnp.float32),
            pltpu.VMEM((SCE,), jnp.int32),
            pltpu.VMEM((SCE,), jnp.int32),
            pltpu.VMEM((CH,), jnp.int32),
            pltpu.VMEM((CH,), jnp.int32),
            pltpu.VMEM((CH,), jnp.int32),
            pltpu.VMEM((CH,), jnp.float32),
            pltpu.VMEM((SLC,), jnp.float32),
            pltpu.SemaphoreType.DMA,
            pltpu.SemaphoreType.DMA,
        ],
    )
    return f(ytab, col, row, et)


def kernel(x, edge_index, edge_type, weights):
    ytab = _make_ytab(x, weights)
    row = edge_index[0]
    col = edge_index[1]
    parts = _make_sc(ytab, col, row, edge_type)
    return _final_add(parts[0, :N], parts[1, :N], ytab[(NET - 1) * N:])


# speedup vs baseline: 6.2122x; 1.0598x over previous
import jax
import jax.numpy as jnp
from jax import lax
from jax.experimental import pallas as pl
from jax.experimental.pallas import tpu as pltpu
from jax.experimental.pallas import tpu_sc as plsc

N = 10000
E = 320000
D = 128
NET = 7
NC = 2
NS = 16
NW = NC * NS
L = 16

CH = 80
EPT = E // NW
SEG = 2000
NSEG = EPT // SEG
CPS = SEG // CH
EPC = E // NS
SCE = 2000
KPS = SCE // CH
NSC1 = EPC // SCE
NB = 70144
SLC = NB // NS
NPAD = 10240
RPT = NPAD // NS
DRB = RPT // CH


def _ytab_body(x_ref, w_ref, o_ref):
    o_ref[...] = jnp.dot(x_ref[...], w_ref[...],
                         preferred_element_type=jnp.float32)


def _make_ytab(x, weights):
    return pl.pallas_call(
        _ytab_body,
        grid=(N // 1000, NET),
        in_specs=[
            pl.BlockSpec((1000, D), lambda i, e: (i, 0)),
            pl.BlockSpec((D, D), lambda i, e: (e, 0)),
        ],
        out_specs=pl.BlockSpec((1000, D), lambda i, e: (e * (N // 1000) + i, 0)),
        out_shape=jax.ShapeDtypeStruct((NET * N, D), jnp.float32),
    )(x, weights)


def _final_body(a_ref, b_ref, c_ref, o_ref):
    o_ref[...] = a_ref[...] + b_ref[...] + c_ref[...]


def _final_add(p0, p1, y6):
    return pl.pallas_call(
        _final_body,
        grid=(N // 1000,),
        in_specs=[pl.BlockSpec((1000, D), lambda i: (i, 0))] * 3,
        out_specs=pl.BlockSpec((1000, D), lambda i: (i, 0)),
        out_shape=jax.ShapeDtypeStruct((N, D), jnp.float32),
    )(p0, p1, y6)


def _sc_body(ytab, col, row, et, parts,
             scale_sh, acc_sh,
             rows2, tblS, bktS, dstS, scl2,
             rowE, etE, bktC, dstCa, dstCb, onesv, cntv,
             sem_g, sem_s):
    core = lax.axis_index("c")
    sid = lax.axis_index("s")
    wid = sid * NC + core

    def _zc16(i, _):
        cntv[pl.ds(i * L, L)] = jnp.zeros((L,), jnp.float32)
        return _
    lax.fori_loop(0, SLC // L, _zc16, None)
    pltpu.sync_copy(cntv, scale_sh.at[pl.ds(sid * SLC, SLC)])

    def _zm(r, _):
        for j in range(D // L):
            rows2[0, r, pl.ds(j * L, L)] = jnp.zeros((L,), jnp.float32)
        return _
    lax.fori_loop(0, CH, _zm, None)
    for b in range(DRB):
        pltpu.sync_copy(rows2.at[0],
                        acc_sh.at[pl.ds(sid * RPT + b * CH, CH), :])

    def _ones16(i, _):
        onesv[pl.ds(i * L, L)] = jnp.ones((L,), jnp.float32)
        return _
    lax.fori_loop(0, CH // L, _ones16, None)
    plsc.subcore_barrier()

    base1 = sid * EPC

    def _p1_iter(s, _):
        off = pl.ds(base1 + s * SCE, SCE)
        pltpu.sync_copy(row.at[off], rowE)
        pltpu.sync_copy(et.at[off], etE)
        def _p1_chunk(k, _):
            def _bk(m, _):
                bktC[pl.ds(m * L, L)] = (rowE[pl.ds(k * CH + m * L, L)] * NET
                                         + etE[pl.ds(k * CH + m * L, L)])
                return _
            lax.fori_loop(0, CH // L, _bk, None)
            pltpu.sync_copy(onesv, scale_sh.at[bktC], add=True)
            return _
        lax.fori_loop(0, KPS, _p1_chunk, None)
        return _
    lax.fori_loop(0, NSC1, _p1_iter, None)
    plsc.subcore_barrier()

    pltpu.sync_copy(scale_sh.at[pl.ds(sid * SLC, SLC)], cntv)

    def _scale16(i, _):
        sl = pl.ds(i * L, L)
        b = sid * SLC + i * L + lax.iota(jnp.int32, L)
        c = cntv[sl]
        s = 1.0 / jnp.maximum(c, 1.0)
        cntv[sl] = jnp.where(b % NET == NET - 1, 0.0, s)
        return _
    lax.fori_loop(0, SLC // L, _scale16, None)
    pltpu.sync_copy(cntv, scale_sh.at[pl.ds(sid * SLC, SLC)])
    plsc.subcore_barrier()

    base2 = wid * EPT

    def _seg_iter(g, _):
        off = pl.ds(base2 + g * SEG, SEG)
        pltpu.sync_copy(col.at[off], tblS)
        pltpu.sync_copy(row.at[off], dstS)
        pltpu.sync_copy(et.at[off], bktS)

        def _cg(m, _):
            sl = pl.ds(m * L, L)
            e16 = bktS[sl]
            tblS[sl] = e16 * N + tblS[sl]
            bktS[sl] = dstS[sl] * NET + e16
            return _
        lax.fori_loop(0, SEG // L, _cg, None)

        def _sg(k, _):
            ck = pl.ds(k * CH, CH)
            pltpu.sync_copy(scale_sh.at[bktS.at[ck]], scl2.at[ck])
            return _
        lax.fori_loop(0, CPS, _sg, None)

        def _scale_rows(i, p):
            def _mul(u, _):
                for v in range(8):
                    e = u * 8 + v
                    sv = lax.broadcast(scl2[pl.ds(i * CH + e, L)][0], (L,))
                    for j in range(D // L):
                        sl = pl.ds(j * L, L)
                        rows2[p, e, sl] = rows2[p, e, sl] * sv
                return _
            lax.fori_loop(0, CH // 8, _mul, None)

        def _stage_dst(i, dstC):
            for m in range(CH // L):
                dstC[pl.ds(m * L, L)] = dstS[pl.ds(i * CH + m * L, L)]

        def _pair_iter(u, _):
            a = 2 * u
            b = a + 1
            ga = pltpu.async_copy(ytab.at[tblS.at[pl.ds(a * CH, CH)]],
                                  rows2.at[0], sem_g)
            gb = pltpu.async_copy(ytab.at[tblS.at[pl.ds(b * CH, CH)]],
                                  rows2.at[1], sem_s)
            ga.wait()
            _stage_dst(a, dstCa)
            _scale_rows(a, 0)
            sa = pltpu.async_copy(rows2.at[0], acc_sh.at[dstCa], sem_g,
                                  add=True)
            gb.wait()
            _stage_dst(b, dstCb)
            _scale_rows(b, 1)
            sa.wait()
            pltpu.sync_copy(rows2.at[1], acc_sh.at[dstCb], add=True)
            return _
        lax.fori_loop(0, CPS // 2, _pair_iter, None)
        gl = pltpu.async_copy(ytab.at[tblS.at[pl.ds((CPS - 1) * CH, CH)]],
                              rows2.at[0], sem_g)
        gl.wait()
        _stage_dst(CPS - 1, dstCa)
        _scale_rows(CPS - 1, 0)
        pltpu.sync_copy(rows2.at[0], acc_sh.at[dstCa], add=True)
        return _
    lax.fori_loop(0, NSEG, _seg_iter, None)
    plsc.subcore_barrier()

    for b in range(DRB):
        off = sid * RPT + b * CH
        pltpu.sync_copy(acc_sh.at[pl.ds(off, CH), :], rows2.at[0])
        pltpu.sync_copy(rows2.at[0], parts.at[core, pl.ds(off, CH), :])


def _make_sc(ytab, col, row, et):
    mesh = plsc.VectorSubcoreMesh(core_axis_name="c", subcore_axis_name="s",
                                  num_cores=NC, num_subcores=NS)
    f = pl.kernel(
        _sc_body,
        out_type=jax.ShapeDtypeStruct((NC, NPAD, D), jnp.float32),
        mesh=mesh,
        scratch_types=[
            pltpu.VMEM_SHARED((NB,), jnp.float32),
            pltpu.VMEM_SHARED((NPAD, D), jnp.float32),
            pltpu.VMEM((2, CH, D), jnp.float32),
            pltpu.VMEM((SEG,), jnp.int32),
            pltpu.VMEM((SEG,), jnp.int32),
            pltpu.VMEM((SEG,), jnp.int32),
            pltpu.VMEM((SEG + L,), jnp.float32),
            pltpu.VMEM((SCE,), jnp.int32),
            pltpu.VMEM((SCE,), jnp.int32),
            pltpu.VMEM((CH,), jnp.int32),
            pltpu.VMEM((CH,), jnp.int32),
            pltpu.VMEM((CH,), jnp.int32),
            pltpu.VMEM((CH,), jnp.float32),
            pltpu.VMEM((SLC,), jnp.float32),
            pltpu.SemaphoreType.DMA,
            pltpu.SemaphoreType.DMA,
        ],
    )
    return f(ytab, col, row, et)


def kernel(x, edge_index, edge_type, weights):
    ytab = _make_ytab(x, weights)
    row = edge_index[0]
    col = edge_index[1]
    parts = _make_sc(ytab, col, row, edge_type)
    return _final_add(parts[0, :N], parts[1, :N], ytab[(NET - 1) * N:])


# speedup vs baseline: 6.4988x; 1.0461x over previous
import jax
import jax.numpy as jnp
from jax import lax
from jax.experimental import pallas as pl
from jax.experimental.pallas import tpu as pltpu
from jax.experimental.pallas import tpu_sc as plsc

N = 10000
E = 320000
D = 128
NET = 7
NC = 2
NS = 16
NW = NC * NS
L = 16

CH = 80
EPT = E // NW
SEG = 2000
NSEG = EPT // SEG
CPS = SEG // CH
EPC = E // NS
SCE = 2000
KPS = SCE // CH
NSC1 = EPC // SCE
NB = 70144
SLC = NB // NS
NPAD = 10240
RPT = NPAD // NS
DRB = RPT // CH


def _ytab_body(x_ref, w_ref, o_ref):
    o_ref[...] = jnp.dot(x_ref[...], w_ref[...],
                         preferred_element_type=jnp.float32)


def _make_ytab(x, weights):
    return pl.pallas_call(
        _ytab_body,
        grid=(N // 1000, NET),
        in_specs=[
            pl.BlockSpec((1000, D), lambda i, e: (i, 0)),
            pl.BlockSpec((D, D), lambda i, e: (e, 0)),
        ],
        out_specs=pl.BlockSpec((1000, D), lambda i, e: (e * (N // 1000) + i, 0)),
        out_shape=jax.ShapeDtypeStruct((NET * N, D), jnp.float32),
    )(x, weights)


def _final_body(a_ref, b_ref, c_ref, o_ref):
    o_ref[...] = a_ref[...] + b_ref[...] + c_ref[...]


def _final_add(p0, p1, y6):
    return pl.pallas_call(
        _final_body,
        grid=(N // 1000,),
        in_specs=[pl.BlockSpec((1000, D), lambda i: (i, 0))] * 3,
        out_specs=pl.BlockSpec((1000, D), lambda i: (i, 0)),
        out_shape=jax.ShapeDtypeStruct((N, D), jnp.float32),
    )(p0, p1, y6)


def _sc_body(ytab, col, row, et, parts,
             scale_sh, acc_sh,
             rows2, tblS, bktS, dstS, scl2,
             rowE, etE, bktC, dstCa, dstCb, onesv, cntv,
             sem_g, sem_s):
    core = lax.axis_index("c")
    sid = lax.axis_index("s")
    wid = sid * NC + core

    def _zc16(i, _):
        cntv[pl.ds(i * L, L)] = jnp.zeros((L,), jnp.float32)
        return _
    lax.fori_loop(0, SLC // L, _zc16, None)
    pltpu.sync_copy(cntv, scale_sh.at[pl.ds(sid * SLC, SLC)])

    def _zm(r, _):
        for j in range(D // L):
            rows2[0, r, pl.ds(j * L, L)] = jnp.zeros((L,), jnp.float32)
        return _
    lax.fori_loop(0, CH, _zm, None)
    for b in range(DRB):
        pltpu.sync_copy(rows2.at[0],
                        acc_sh.at[pl.ds(sid * RPT + b * CH, CH), :])

    def _ones16(i, _):
        onesv[pl.ds(i * L, L)] = jnp.ones((L,), jnp.float32)
        return _
    lax.fori_loop(0, CH // L, _ones16, None)
    plsc.subcore_barrier()

    base1 = sid * EPC

    def _p1_iter(s, _):
        off = pl.ds(base1 + s * SCE, SCE)
        pltpu.sync_copy(row.at[off], rowE)
        pltpu.sync_copy(et.at[off], etE)
        def _bk(k, dst):
            def _b1(m, _):
                dst[pl.ds(m * L, L)] = (rowE[pl.ds(k * CH + m * L, L)] * NET
                                        + etE[pl.ds(k * CH + m * L, L)])
                return _
            lax.fori_loop(0, CH // L, _b1, None)

        def _p1_pair(u, _):
            a = 2 * u
            _bk(a, bktC)
            da = pltpu.async_copy(onesv, scale_sh.at[bktC], sem_g, add=True)
            _bk(a + 1, dstCa)
            db = pltpu.async_copy(onesv, scale_sh.at[dstCa], sem_s, add=True)
            da.wait()
            db.wait()
            return _
        lax.fori_loop(0, KPS // 2, _p1_pair, None)
        _bk(KPS - 1, bktC)
        pltpu.sync_copy(onesv, scale_sh.at[bktC], add=True)
        return _
    lax.fori_loop(0, NSC1, _p1_iter, None)
    plsc.subcore_barrier()

    pltpu.sync_copy(scale_sh.at[pl.ds(sid * SLC, SLC)], cntv)

    def _scale16(i, _):
        sl = pl.ds(i * L, L)
        b = sid * SLC + i * L + lax.iota(jnp.int32, L)
        c = cntv[sl]
        s = 1.0 / jnp.maximum(c, 1.0)
        cntv[sl] = jnp.where(b % NET == NET - 1, 0.0, s)
        return _
    lax.fori_loop(0, SLC // L, _scale16, None)
    pltpu.sync_copy(cntv, scale_sh.at[pl.ds(sid * SLC, SLC)])
    plsc.subcore_barrier()

    base2 = wid * EPT

    def _seg_iter(g, _):
        off = pl.ds(base2 + g * SEG, SEG)
        pltpu.sync_copy(col.at[off], tblS)
        pltpu.sync_copy(row.at[off], dstS)
        pltpu.sync_copy(et.at[off], bktS)

        def _cg(m, _):
            sl = pl.ds(m * L, L)
            e16 = bktS[sl]
            tblS[sl] = e16 * N + tblS[sl]
            bktS[sl] = dstS[sl] * NET + e16
            return _
        lax.fori_loop(0, SEG // L, _cg, None)

        def _sg(u, _):
            ca = pl.ds((2 * u) * CH, CH)
            cb = pl.ds((2 * u + 1) * CH, CH)
            da = pltpu.async_copy(scale_sh.at[bktS.at[ca]], scl2.at[ca],
                                  sem_g)
            db = pltpu.async_copy(scale_sh.at[bktS.at[cb]], scl2.at[cb],
                                  sem_s)
            da.wait()
            db.wait()
            return _
        lax.fori_loop(0, CPS // 2, _sg, None)
        cl = pl.ds((CPS - 1) * CH, CH)
        pltpu.sync_copy(scale_sh.at[bktS.at[cl]], scl2.at[cl])

        def _scale_rows(i, p):
            def _mul(u, _):
                for v in range(8):
                    e = u * 8 + v
                    sv = lax.broadcast(scl2[pl.ds(i * CH + e, L)][0], (L,))
                    for j in range(D // L):
                        sl = pl.ds(j * L, L)
                        rows2[p, e, sl] = rows2[p, e, sl] * sv
                return _
            lax.fori_loop(0, CH // 8, _mul, None)

        def _stage_dst(i, dstC):
            for m in range(CH // L):
                dstC[pl.ds(m * L, L)] = dstS[pl.ds(i * CH + m * L, L)]

        def _pair_iter(u, _):
            a = 2 * u
            b = a + 1
            ga = pltpu.async_copy(ytab.at[tblS.at[pl.ds(a * CH, CH)]],
                                  rows2.at[0], sem_g)
            gb = pltpu.async_copy(ytab.at[tblS.at[pl.ds(b * CH, CH)]],
                                  rows2.at[1], sem_s)
            ga.wait()
            _stage_dst(a, dstCa)
            _scale_rows(a, 0)
            sa = pltpu.async_copy(rows2.at[0], acc_sh.at[dstCa], sem_g,
                                  add=True)
            gb.wait()
            _stage_dst(b, dstCb)
            _scale_rows(b, 1)
            sa.wait()
            pltpu.sync_copy(rows2.at[1], acc_sh.at[dstCb], add=True)
            return _
        lax.fori_loop(0, CPS // 2, _pair_iter, None)
        gl = pltpu.async_copy(ytab.at[tblS.at[pl.ds((CPS - 1) * CH, CH)]],
                              rows2.at[0], sem_g)
        gl.wait()
        _stage_dst(CPS - 1, dstCa)
        _scale_rows(CPS - 1, 0)
        pltpu.sync_copy(rows2.at[0], acc_sh.at[dstCa], add=True)
        return _
    lax.fori_loop(0, NSEG, _seg_iter, None)
    plsc.subcore_barrier()

    for b in range(DRB):
        off = sid * RPT + b * CH
        pltpu.sync_copy(acc_sh.at[pl.ds(off, CH), :], rows2.at[0])
        pltpu.sync_copy(rows2.at[0], parts.at[core, pl.ds(off, CH), :])


def _make_sc(ytab, col, row, et):
    mesh = plsc.VectorSubcoreMesh(core_axis_name="c", subcore_axis_name="s",
                                  num_cores=NC, num_subcores=NS)
    f = pl.kernel(
        _sc_body,
        out_type=jax.ShapeDtypeStruct((NC, NPAD, D), jnp.float32),
        mesh=mesh,
        scratch_types=[
            pltpu.VMEM_SHARED((NB,), jnp.float32),
            pltpu.VMEM_SHARED((NPAD, D), jnp.float32),
            pltpu.VMEM((2, CH, D), jnp.float32),
            pltpu.VMEM((SEG,), jnp.int32),
            pltpu.VMEM((SEG,), jnp.int32),
            pltpu.VMEM((SEG,), jnp.int32),
            pltpu.VMEM((SEG + L,), jnp.float32),
            pltpu.VMEM((SCE,), jnp.int32),
            pltpu.VMEM((SCE,), jnp.int32),
            pltpu.VMEM((CH,), jnp.int32),
            pltpu.VMEM((CH,), jnp.int32),
            pltpu.VMEM((CH,), jnp.int32),
            pltpu.VMEM((CH,), jnp.float32),
            pltpu.VMEM((SLC,), jnp.float32),
            pltpu.SemaphoreType.DMA,
            pltpu.SemaphoreType.DMA,
        ],
    )
    return f(ytab, col, row, et)


def kernel(x, edge_index, edge_type, weights):
    ytab = _make_ytab(x, weights)
    row = edge_index[0]
    col = edge_index[1]
    parts = _make_sc(ytab, col, row, edge_type)
    return _final_add(parts[0, :N], parts[1, :N], ytab[(NET - 1) * N:])


# speedup vs baseline: 6.7318x; 1.0359x over previous
import jax
import jax.numpy as jnp
from jax import lax
from jax.experimental import pallas as pl
from jax.experimental.pallas import tpu as pltpu
from jax.experimental.pallas import tpu_sc as plsc

N = 10000
E = 320000
D = 128
NET = 7
NC = 2
NS = 16
NW = NC * NS
L = 16

CH = 80
EPT = E // NW
SEG = 2000
NSEG = EPT // SEG
CPS = SEG // CH
EPC = E // NS
SCE = 2000
KPS = SCE // CH
NSC1 = EPC // SCE
NB = 70144
SLC = NB // NS
NPAD = 10240
RPT = NPAD // NS
DRB = RPT // CH


def _ytab_body(x_ref, w_ref, o_ref):
    o_ref[...] = jnp.dot(x_ref[...], w_ref[...],
                         preferred_element_type=jnp.float32)


def _make_ytab(x, weights):
    return pl.pallas_call(
        _ytab_body,
        grid=(N // 1000, NET),
        in_specs=[
            pl.BlockSpec((1000, D), lambda i, e: (i, 0)),
            pl.BlockSpec((D, D), lambda i, e: (e, 0)),
        ],
        out_specs=pl.BlockSpec((1000, D), lambda i, e: (e * (N // 1000) + i, 0)),
        out_shape=jax.ShapeDtypeStruct((NET * N, D), jnp.float32),
    )(x, weights)


def _final_body(a_ref, b_ref, c_ref, o_ref):
    o_ref[...] = a_ref[...] + b_ref[...] + c_ref[...]


def _final_add(p0, p1, y6):
    return pl.pallas_call(
        _final_body,
        grid=(N // 1000,),
        in_specs=[pl.BlockSpec((1000, D), lambda i: (i, 0))] * 3,
        out_specs=pl.BlockSpec((1000, D), lambda i: (i, 0)),
        out_shape=jax.ShapeDtypeStruct((N, D), jnp.float32),
    )(p0, p1, y6)


def _sc_body(ytab, col, row, et, parts,
             scale_sh, acc_sh,
             rows2, tblS, bktS, dstS, scl2,
             rowE, etE, bktC, dstCa, dstCb, onesv, cntv,
             sem_g, sem_s):
    core = lax.axis_index("c")
    sid = lax.axis_index("s")
    wid = sid * NC + core

    def _zc16(i, _):
        cntv[pl.ds(i * L, L)] = jnp.zeros((L,), jnp.float32)
        return _
    lax.fori_loop(0, SLC // L, _zc16, None)
    pltpu.sync_copy(cntv, scale_sh.at[pl.ds(sid * SLC, SLC)])

    def _zm(r, _):
        for j in range(D // L):
            rows2[0, r, pl.ds(j * L, L)] = jnp.zeros((L,), jnp.float32)
        return _
    lax.fori_loop(0, CH, _zm, None)
    for b in range(DRB):
        pltpu.sync_copy(rows2.at[0],
                        acc_sh.at[pl.ds(sid * RPT + b * CH, CH), :])

    def _ones16(i, _):
        onesv[pl.ds(i * L, L)] = jnp.ones((L,), jnp.float32)
        return _
    lax.fori_loop(0, CH // L, _ones16, None)
    plsc.subcore_barrier()

    base1 = sid * EPC

    def _p1_iter(s, _):
        off = pl.ds(base1 + s * SCE, SCE)
        d1 = pltpu.async_copy(row.at[off], rowE, sem_g)
        d2 = pltpu.async_copy(et.at[off], etE, sem_s)
        d1.wait()
        d2.wait()
        def _bk(k, dst):
            def _b1(m, _):
                dst[pl.ds(m * L, L)] = (rowE[pl.ds(k * CH + m * L, L)] * NET
                                        + etE[pl.ds(k * CH + m * L, L)])
                return _
            lax.fori_loop(0, CH // L, _b1, None)

        def _p1_pair(u, _):
            a = 2 * u
            _bk(a, bktC)
            da = pltpu.async_copy(onesv, scale_sh.at[bktC], sem_g, add=True)
            _bk(a + 1, dstCa)
            db = pltpu.async_copy(onesv, scale_sh.at[dstCa], sem_s, add=True)
            da.wait()
            db.wait()
            return _
        lax.fori_loop(0, KPS // 2, _p1_pair, None)
        _bk(KPS - 1, bktC)
        pltpu.sync_copy(onesv, scale_sh.at[bktC], add=True)
        return _
    lax.fori_loop(0, NSC1, _p1_iter, None)
    plsc.subcore_barrier()

    pltpu.sync_copy(scale_sh.at[pl.ds(sid * SLC, SLC)], cntv)

    def _scale16(i, _):
        sl = pl.ds(i * L, L)
        b = sid * SLC + i * L + lax.iota(jnp.int32, L)
        c = cntv[sl]
        s = 1.0 / jnp.maximum(c, 1.0)
        cntv[sl] = jnp.where(b % NET == NET - 1, 0.0, s)
        return _
    lax.fori_loop(0, SLC // L, _scale16, None)
    pltpu.sync_copy(cntv, scale_sh.at[pl.ds(sid * SLC, SLC)])
    plsc.subcore_barrier()

    base2 = wid * EPT

    def _seg_iter(g, _):
        off = pl.ds(base2 + g * SEG, SEG)
        d1 = pltpu.async_copy(col.at[off], tblS, sem_g)
        d2 = pltpu.async_copy(row.at[off], dstS, sem_s)
        d3 = pltpu.async_copy(et.at[off], bktS, sem_g)
        d1.wait()
        d2.wait()
        d3.wait()

        def _cg(m, _):
            sl = pl.ds(m * L, L)
            e16 = bktS[sl]
            tblS[sl] = e16 * N + tblS[sl]
            bktS[sl] = dstS[sl] * NET + e16
            return _
        lax.fori_loop(0, SEG // L, _cg, None)

        def _sg(u, _):
            ca = pl.ds((2 * u) * CH, CH)
            cb = pl.ds((2 * u + 1) * CH, CH)
            da = pltpu.async_copy(scale_sh.at[bktS.at[ca]], scl2.at[ca],
                                  sem_g)
            db = pltpu.async_copy(scale_sh.at[bktS.at[cb]], scl2.at[cb],
                                  sem_s)
            da.wait()
            db.wait()
            return _
        lax.fori_loop(0, CPS // 2, _sg, None)
        cl = pl.ds((CPS - 1) * CH, CH)
        pltpu.sync_copy(scale_sh.at[bktS.at[cl]], scl2.at[cl])

        def _scale_rows(i, p):
            def _mul(u, _):
                for v in range(8):
                    e = u * 8 + v
                    sv = lax.broadcast(scl2[pl.ds(i * CH + e, L)][0], (L,))
                    for j in range(D // L):
                        sl = pl.ds(j * L, L)
                        rows2[p, e, sl] = rows2[p, e, sl] * sv
                return _
            lax.fori_loop(0, CH // 8, _mul, None)

        def _stage_dst(i, dstC):
            for m in range(CH // L):
                dstC[pl.ds(m * L, L)] = dstS[pl.ds(i * CH + m * L, L)]

        def _pair_iter(u, _):
            a = 2 * u
            b = a + 1
            ga = pltpu.async_copy(ytab.at[tblS.at[pl.ds(a * CH, CH)]],
                                  rows2.at[0], sem_g)
            gb = pltpu.async_copy(ytab.at[tblS.at[pl.ds(b * CH, CH)]],
                                  rows2.at[1], sem_s)
            ga.wait()
            _stage_dst(a, dstCa)
            _scale_rows(a, 0)
            sa = pltpu.async_copy(rows2.at[0], acc_sh.at[dstCa], sem_g,
                                  add=True)
            gb.wait()
            _stage_dst(b, dstCb)
            _scale_rows(b, 1)
            sa.wait()
            pltpu.sync_copy(rows2.at[1], acc_sh.at[dstCb], add=True)
            return _
        lax.fori_loop(0, CPS // 2, _pair_iter, None)
        gl = pltpu.async_copy(ytab.at[tblS.at[pl.ds((CPS - 1) * CH, CH)]],
                              rows2.at[0], sem_g)
        gl.wait()
        _stage_dst(CPS - 1, dstCa)
        _scale_rows(CPS - 1, 0)
        pltpu.sync_copy(rows2.at[0], acc_sh.at[dstCa], add=True)
        return _
    lax.fori_loop(0, NSEG, _seg_iter, None)
    plsc.subcore_barrier()

    off0 = sid * RPT
    pltpu.sync_copy(acc_sh.at[pl.ds(off0, CH), :], rows2.at[0])
    for b in range(DRB):
        off = sid * RPT + b * CH
        p = b % 2
        q = 1 - p
        do = pltpu.async_copy(rows2.at[p], parts.at[core, pl.ds(off, CH), :],
                              sem_g)
        if b < DRB - 1:
            pltpu.sync_copy(acc_sh.at[pl.ds(off + CH, CH), :], rows2.at[q])
        do.wait()


def _make_sc(ytab, col, row, et):
    mesh = plsc.VectorSubcoreMesh(core_axis_name="c", subcore_axis_name="s",
                                  num_cores=NC, num_subcores=NS)
    f = pl.kernel(
        _sc_body,
        out_type=jax.ShapeDtypeStruct((NC, NPAD, D), jnp.float32),
        mesh=mesh,
        scratch_types=[
            pltpu.VMEM_SHARED((NB,), jnp.float32),
            pltpu.VMEM_SHARED((NPAD, D), jnp.float32),
            pltpu.VMEM((2, CH, D), jnp.float32),
            pltpu.VMEM((SEG,), jnp.int32),
            pltpu.VMEM((SEG,), jnp.int32),
            pltpu.VMEM((SEG,), jnp.int32),
            pltpu.VMEM((SEG + L,), jnp.float32),
            pltpu.VMEM((SCE,), jnp.int32),
            pltpu.VMEM((SCE,), jnp.int32),
            pltpu.VMEM((CH,), jnp.int32),
            pltpu.VMEM((CH,), jnp.int32),
            pltpu.VMEM((CH,), jnp.int32),
            pltpu.VMEM((CH,), jnp.float32),
            pltpu.VMEM((SLC,), jnp.float32),
            pltpu.SemaphoreType.DMA,
            pltpu.SemaphoreType.DMA,
        ],
    )
    return f(ytab, col, row, et)


def kernel(x, edge_index, edge_type, weights):
    ytab = _make_ytab(x, weights)
    row = edge_index[0]
    col = edge_index[1]
    parts = _make_sc(ytab, col, row, edge_type)
    return _final_add(parts[0, :N], parts[1, :N], ytab[(NET - 1) * N:])


# speedup vs baseline: 6.7448x; 1.0019x over previous
import jax
import jax.numpy as jnp
from jax import lax
from jax.experimental import pallas as pl
from jax.experimental.pallas import tpu as pltpu
from jax.experimental.pallas import tpu_sc as plsc

N = 10000
E = 320000
D = 128
NET = 7
NC = 2
NS = 16
NW = NC * NS
L = 16

CH = 80
EPT = E // NW
SEG = 2000
NSEG = EPT // SEG
CPS = SEG // CH
EPC = E // NS
SCE = 2000
KPS = SCE // CH
NSC1 = EPC // SCE
NB = 70144
SLC = NB // NS
NPAD = 10240
RPT = NPAD // NS
DRB = RPT // CH


def _ytab_body(x_ref, w_ref, o_ref):
    o_ref[...] = jnp.dot(x_ref[...], w_ref[...],
                         preferred_element_type=jnp.float32)


def _make_ytab(x, weights):
    return pl.pallas_call(
        _ytab_body,
        grid=(N // 1000, NET),
        in_specs=[
            pl.BlockSpec((1000, D), lambda i, e: (i, 0)),
            pl.BlockSpec((D, D), lambda i, e: (e, 0)),
        ],
        out_specs=pl.BlockSpec((1000, D), lambda i, e: (e * (N // 1000) + i, 0)),
        out_shape=jax.ShapeDtypeStruct((NET * N, D), jnp.float32),
    )(x, weights)


def _final_body(a_ref, b_ref, c_ref, o_ref):
    o_ref[...] = a_ref[...] + b_ref[...] + c_ref[...]


def _final_add(p0, p1, y6):
    return pl.pallas_call(
        _final_body,
        grid=(N // 1000,),
        in_specs=[pl.BlockSpec((1000, D), lambda i: (i, 0))] * 3,
        out_specs=pl.BlockSpec((1000, D), lambda i: (i, 0)),
        out_shape=jax.ShapeDtypeStruct((N, D), jnp.float32),
    )(p0, p1, y6)


def _sc_body(ytab, col, row, et, parts,
             scale_sh, acc_sh,
             rows2, tblS, bktS, dstS, scl2,
             rowE, etE, bktC, dstCa, dstCb, onesv, cntv,
             sem_g, sem_s):
    core = lax.axis_index("c")
    sid = lax.axis_index("s")
    wid = sid * NC + core

    def _zc16(i, _):
        cntv[pl.ds(i * L, L)] = jnp.zeros((L,), jnp.float32)
        return _
    lax.fori_loop(0, SLC // L, _zc16, None)
    pltpu.sync_copy(cntv, scale_sh.at[pl.ds(sid * SLC, SLC)])

    def _zm(r, _):
        for j in range(D // L):
            rows2[0, r, pl.ds(j * L, L)] = jnp.zeros((L,), jnp.float32)
        return _
    lax.fori_loop(0, CH, _zm, None)
    for b in range(DRB):
        pltpu.sync_copy(rows2.at[0],
                        acc_sh.at[pl.ds(sid * RPT + b * CH, CH), :])

    def _ones16(i, _):
        onesv[pl.ds(i * L, L)] = jnp.ones((L,), jnp.float32)
        return _
    lax.fori_loop(0, CH // L, _ones16, None)
    plsc.subcore_barrier()

    base1 = sid * EPC

    def _p1_iter(s, _):
        off = pl.ds(base1 + s * SCE, SCE)
        d1 = pltpu.async_copy(row.at[off], rowE, sem_g)
        d2 = pltpu.async_copy(et.at[off], etE, sem_s)
        d1.wait()
        d2.wait()
        def _bk(k, dst):
            def _b1(m, _):
                dst[pl.ds(m * L, L)] = (rowE[pl.ds(k * CH + m * L, L)] * NET
                                        + etE[pl.ds(k * CH + m * L, L)])
                return _
            lax.fori_loop(0, CH // L, _b1, None)

        def _p1_pair(u, _):
            a = 2 * u
            _bk(a, bktC)
            da = pltpu.async_copy(onesv, scale_sh.at[bktC], sem_g, add=True)
            _bk(a + 1, dstCa)
            db = pltpu.async_copy(onesv, scale_sh.at[dstCa], sem_s, add=True)
            da.wait()
            db.wait()
            return _
        lax.fori_loop(0, KPS // 2, _p1_pair, None)
        _bk(KPS - 1, bktC)
        pltpu.sync_copy(onesv, scale_sh.at[bktC], add=True)
        return _
    lax.fori_loop(0, NSC1, _p1_iter, None)
    plsc.subcore_barrier()

    pltpu.sync_copy(scale_sh.at[pl.ds(sid * SLC, SLC)], cntv)

    def _scale16(i, _):
        sl = pl.ds(i * L, L)
        b = sid * SLC + i * L + lax.iota(jnp.int32, L)
        c = cntv[sl]
        s = 1.0 / jnp.maximum(c, 1.0)
        cntv[sl] = jnp.where(b % NET == NET - 1, 0.0, s)
        return _
    lax.fori_loop(0, SLC // L, _scale16, None)
    pltpu.sync_copy(cntv, scale_sh.at[pl.ds(sid * SLC, SLC)])
    plsc.subcore_barrier()

    base2 = wid * EPT

    def _seg_iter(g, _):
        off = pl.ds(base2 + g * SEG, SEG)
        d1 = pltpu.async_copy(col.at[off], tblS, sem_g)
        d2 = pltpu.async_copy(row.at[off], dstS, sem_s)
        d3 = pltpu.async_copy(et.at[off], bktS, sem_g)
        d1.wait()
        d2.wait()
        d3.wait()

        def _cg(m, _):
            sl = pl.ds(m * L, L)
            e16 = bktS[sl]
            tblS[sl] = e16 * N + tblS[sl]
            bktS[sl] = dstS[sl] * NET + e16
            return _
        lax.fori_loop(0, SEG // L, _cg, None)

        def _sg(u, _):
            ca = pl.ds((2 * u) * CH, CH)
            cb = pl.ds((2 * u + 1) * CH, CH)
            da = pltpu.async_copy(scale_sh.at[bktS.at[ca]], scl2.at[ca],
                                  sem_g)
            db = pltpu.async_copy(scale_sh.at[bktS.at[cb]], scl2.at[cb],
                                  sem_s)
            da.wait()
            db.wait()
            return _
        lax.fori_loop(0, CPS // 2, _sg, None)
        cl = pl.ds((CPS - 1) * CH, CH)
        pltpu.sync_copy(scale_sh.at[bktS.at[cl]], scl2.at[cl])

        def _scale_rows(i, p):
            def _mul(u, _):
                for v in range(16):
                    e = u * 16 + v
                    sv = lax.broadcast(scl2[pl.ds(i * CH + e, L)][0], (L,))
                    for j in range(D // L):
                        sl = pl.ds(j * L, L)
                        rows2[p, e, sl] = rows2[p, e, sl] * sv
                return _
            lax.fori_loop(0, CH // 16, _mul, None)

        def _stage_dst(i, dstC):
            for m in range(CH // L):
                dstC[pl.ds(m * L, L)] = dstS[pl.ds(i * CH + m * L, L)]

        def _pair_iter(u, _):
            a = 2 * u
            b = a + 1
            ga = pltpu.async_copy(ytab.at[tblS.at[pl.ds(a * CH, CH)]],
                                  rows2.at[0], sem_g)
            gb = pltpu.async_copy(ytab.at[tblS.at[pl.ds(b * CH, CH)]],
                                  rows2.at[1], sem_s)
            ga.wait()
            _stage_dst(a, dstCa)
            _scale_rows(a, 0)
            sa = pltpu.async_copy(rows2.at[0], acc_sh.at[dstCa], sem_g,
                                  add=True)
            gb.wait()
            _stage_dst(b, dstCb)
            _scale_rows(b, 1)
            sa.wait()
            pltpu.sync_copy(rows2.at[1], acc_sh.at[dstCb], add=True)
            return _
        lax.fori_loop(0, CPS // 2, _pair_iter, None)
        gl = pltpu.async_copy(ytab.at[tblS.at[pl.ds((CPS - 1) * CH, CH)]],
                              rows2.at[0], sem_g)
        gl.wait()
        _stage_dst(CPS - 1, dstCa)
        _scale_rows(CPS - 1, 0)
        pltpu.sync_copy(rows2.at[0], acc_sh.at[dstCa], add=True)
        return _
    lax.fori_loop(0, NSEG, _seg_iter, None)
    plsc.subcore_barrier()

    off0 = sid * RPT
    pltpu.sync_copy(acc_sh.at[pl.ds(off0, CH), :], rows2.at[0])
    for b in range(DRB):
        off = sid * RPT + b * CH
        p = b % 2
        q = 1 - p
        do = pltpu.async_copy(rows2.at[p], parts.at[core, pl.ds(off, CH), :],
                              sem_g)
        if b < DRB - 1:
            pltpu.sync_copy(acc_sh.at[pl.ds(off + CH, CH), :], rows2.at[q])
        do.wait()


def _make_sc(ytab, col, row, et):
    mesh = plsc.VectorSubcoreMesh(core_axis_name="c", subcore_axis_name="s",
                                  num_cores=NC, num_subcores=NS)
    f = pl.kernel(
        _sc_body,
        out_type=jax.ShapeDtypeStruct((NC, NPAD, D), jnp.float32),
        mesh=mesh,
        scratch_types=[
            pltpu.VMEM_SHARED((NB,), jnp.float32),
            pltpu.VMEM_SHARED((NPAD, D), jnp.float32),
            pltpu.VMEM((2, CH, D), jnp.float32),
            pltpu.VMEM((SEG,), jnp.int32),
            pltpu.VMEM((SEG,), jnp.int32),
            pltpu.VMEM((SEG,), jnp.int32),
            pltpu.VMEM((SEG + L,), jnp.float32),
            pltpu.VMEM((SCE,), jnp.int32),
            pltpu.VMEM((SCE,), jnp.int32),
            pltpu.VMEM((CH,), jnp.int32),
            pltpu.VMEM((CH,), jnp.int32),
            pltpu.VMEM((CH,), jnp.int32),
            pltpu.VMEM((CH,), jnp.float32),
            pltpu.VMEM((SLC,), jnp.float32),
            pltpu.SemaphoreType.DMA,
            pltpu.SemaphoreType.DMA,
        ],
    )
    return f(ytab, col, row, et)


def kernel(x, edge_index, edge_type, weights):
    ytab = _make_ytab(x, weights)
    row = edge_index[0]
    col = edge_index[1]
    parts = _make_sc(ytab, col, row, edge_type)
    return _final_add(parts[0, :N], parts[1, :N], ytab[(NET - 1) * N:])
